# Initial kernel scaffold; baseline (speedup 1.0000x reference)
import jax
import jax.numpy as jnp
from jax.experimental import pallas as pl


def kernel(x, edge_index, W0_1, W1_1, b1, W0_2, W1_2, b2):
    raise NotImplementedError("write your pallas kernel here")


# speedup vs baseline: 7.8071x; 7.8071x over previous
import functools
import jax
import jax.numpy as jnp
from jax import lax
from jax.experimental import pallas as pl
from jax.experimental.pallas import tpu as pltpu
from jax.experimental.pallas import tpu_sc as plsc

N = 10000
F = 128
NC, NS, L = 2, 16, 16
NW = NC * NS
NPAD = 10240
RPT = NPAD // NS
EPW = 10240
EPAD = NW * EPW
CH = 128
NCHUNK = EPW // CH
BLK = 1024
GRID = NPAD // BLK

_MESH = plsc.VectorSubcoreMesh(
    core_axis_name="c", subcore_axis_name="s", num_cores=NC, num_subcores=NS
)


def _deg_body(src_hbm, ones_hbm, out_hbm, shared, sidx, onesb, zbuf):
    c = lax.axis_index("c")
    s = lax.axis_index("s")
    w = c * NS + s
    pltpu.sync_copy(ones_hbm.at[pl.ds(0, CH)], onesb)
    pltpu.sync_copy(ones_hbm.at[pl.ds(CH, CH)], zbuf)
    for k in range(RPT // CH):
        pltpu.sync_copy(zbuf, shared.at[pl.ds(s * RPT + k * CH, CH)])
    plsc.subcore_barrier()

    pltpu.sync_copy(src_hbm.at[w], sidx)

    def body(j, _):
        pltpu.sync_copy(onesb, shared.at[sidx.at[j]], add=True)
        return 0

    lax.fori_loop(0, NCHUNK, body, 0)
    plsc.subcore_barrier()

    for k in range(RPT // CH):
        pltpu.sync_copy(shared.at[pl.ds(s * RPT + k * CH, CH)], zbuf)
        pltpu.sync_copy(zbuf, out_hbm.at[c, pl.ds(s * RPT + k * CH, CH)])


def _scatter_body(z_hbm, src_hbm, dst_hbm, out_hbm, shared, sidx, didx, gbuf, sem):
    c = lax.axis_index("c")
    s = lax.axis_index("s")
    w = c * NS + s

    pltpu.sync_copy(z_hbm.at[pl.ds(NPAD - CH, CH)], gbuf)
    for k in range(RPT // CH):
        pltpu.sync_copy(gbuf, shared.at[pl.ds(s * RPT + k * CH, CH)])
    plsc.subcore_barrier()

    pltpu.sync_copy(src_hbm.at[w], sidx)
    pltpu.sync_copy(dst_hbm.at[w], didx)

    def body(j, _):
        pltpu.async_copy(z_hbm.at[sidx.at[j]], gbuf, sem).wait()
        pltpu.sync_copy(gbuf, shared.at[didx.at[j]], add=True)
        return 0

    lax.fori_loop(0, NCHUNK, body, 0)
    plsc.subcore_barrier()

    for k in range(RPT // CH):
        pltpu.sync_copy(shared.at[pl.ds(s * RPT + k * CH, CH)], gbuf)
        pltpu.sync_copy(gbuf, out_hbm.at[c, pl.ds(s * RPT + k * CH, CH)])


_deg_kernel = functools.partial(
    pl.kernel,
    out_type=jax.ShapeDtypeStruct((NC, NPAD, F), jnp.float32),
    mesh=_MESH,
    scratch_types=[
        pltpu.VMEM_SHARED((NPAD, F), jnp.float32),
        pltpu.VMEM((NCHUNK, CH), jnp.int32),
        pltpu.VMEM((CH, F), jnp.float32),
        pltpu.VMEM((CH, F), jnp.float32),
    ],
)(_deg_body)

_scatter_kernel = functools.partial(
    pl.kernel,
    out_type=jax.ShapeDtypeStruct((NC, NPAD, F), jnp.float32),
    mesh=_MESH,
    scratch_types=[
        pltpu.VMEM_SHARED((NPAD, F), jnp.float32),
        pltpu.VMEM((NCHUNK, CH), jnp.int32),
        pltpu.VMEM((NCHUNK, CH), jnp.int32),
        pltpu.VMEM((CH, F), jnp.float32),
        pltpu.SemaphoreType.DMA,
    ],
)(_scatter_body)


def _col0(m):
    e0 = (lax.broadcasted_iota(jnp.int32, (F, 1), 0) == 0).astype(jnp.float32)
    return lax.dot_general(m, e0, (((1,), (0,)), ((), ())),
                           preferred_element_type=jnp.float32)


def _pre_body(dega_ref, degb_ref, x_ref, w0_ref, w1_ref,
              xw0_ref, z1_ref, dinv_ref):
    deg = _col0(dega_ref[...] + degb_ref[...])
    rows = lax.broadcasted_iota(jnp.int32, (BLK, 1), 0) + pl.program_id(0) * BLK
    valid = (deg > 0.0) & (rows < N)
    dinv = jnp.where(valid, lax.rsqrt(jnp.maximum(deg, 1.0)), 0.0)
    x = x_ref[...]
    xw0_ref[...] = jnp.dot(x, w0_ref[...], preferred_element_type=jnp.float32)
    z1_ref[...] = jnp.dot(dinv * x, w1_ref[...], preferred_element_type=jnp.float32)
    dinv_ref[...] = dinv


def _mid_body(xw0_ref, ga_ref, gb_ref, dinv_ref, b_ref, w0_ref, w1_ref,
              h1_ref, hw0_ref, z2_ref):
    dinv = dinv_ref[...]
    h1 = jnp.maximum(
        xw0_ref[...] - dinv * (ga_ref[...] + gb_ref[...]) + b_ref[...], 0.0)
    h1_ref[...] = h1
    hw0_ref[...] = jnp.dot(h1, w0_ref[...], preferred_element_type=jnp.float32)
    z2_ref[...] = jnp.dot(dinv * h1, w1_ref[...], preferred_element_type=jnp.float32)


def _post_body(h1_ref, hw0_ref, ga_ref, gb_ref, dinv_ref, b_ref, out_ref):
    h2 = jnp.maximum(
        hw0_ref[...] - dinv_ref[...] * (ga_ref[...] + gb_ref[...]) + b_ref[...], 0.0)
    out_ref[...] = (h1_ref[...] + h2) * 0.5


_row = pl.BlockSpec((BLK, F), lambda i: (i, 0))
_rowcol = pl.BlockSpec((BLK, 1), lambda i: (i, 0))
_rowdeg = pl.BlockSpec((BLK, F), lambda i: (i, 0))
_wspec = pl.BlockSpec((F, F), lambda i: (0, 0))
_bspec = pl.BlockSpec((1, F), lambda i: (0, 0))

_mat = jax.ShapeDtypeStruct((NPAD, F), jnp.float32)
_colv = jax.ShapeDtypeStruct((NPAD, 1), jnp.float32)

_pre_call = pl.pallas_call(
    _pre_body,
    grid=(GRID,),
    in_specs=[_rowdeg, _rowdeg, _row, _wspec, _wspec],
    out_specs=[_row, _row, _rowcol],
    out_shape=[_mat, _mat, _colv],
)

_mid_call = pl.pallas_call(
    _mid_body,
    grid=(GRID,),
    in_specs=[_row, _row, _row, _rowcol, _bspec, _wspec, _wspec],
    out_specs=[_row, _row, _row],
    out_shape=[_mat, _mat, _mat],
)

_post_call = pl.pallas_call(
    _post_body,
    grid=(GRID,),
    in_specs=[_row, _row, _row, _row, _rowcol, _bspec],
    out_specs=_row,
    out_shape=_mat,
)


def kernel(x, edge_index, W0_1, W1_1, b1, W0_2, W1_2, b2):
    src = edge_index[0]
    dst = edge_index[1]
    e = src.shape[0]
    pad_e = EPAD - e
    srcp = jnp.concatenate([src, jnp.full((pad_e,), N, jnp.int32)])
    dstp = jnp.concatenate([dst, jnp.full((pad_e,), NPAD - 1, jnp.int32)])
    src_b = srcp.reshape(NW, NCHUNK, CH)
    dst_b = dstp.reshape(NW, NCHUNK, CH)
    xp = jnp.concatenate([x, jnp.zeros((NPAD - N, F), jnp.float32)], axis=0)

    ones_c = jnp.zeros((2 * CH, F), jnp.float32).at[:CH, 0].set(1.0)
    degp = _deg_kernel(src_b, ones_c)
    xw0, z1, dinv = _pre_call(degp[0], degp[1], xp, W0_1, W1_1)
    g1 = _scatter_kernel(z1, src_b, dst_b)
    h1, hw0, z2 = _mid_call(xw0, g1[0], g1[1], dinv, b1.reshape(1, F), W0_2, W1_2)
    g2 = _scatter_kernel(z2, src_b, dst_b)
    out = _post_call(h1, hw0, g2[0], g2[1], dinv, b2.reshape(1, F))
    return out[:N]


# speedup vs baseline: 8.5045x; 1.0893x over previous
import functools
import jax
import jax.numpy as jnp
from jax import lax
from jax.experimental import pallas as pl
from jax.experimental.pallas import tpu as pltpu
from jax.experimental.pallas import tpu_sc as plsc

N = 10000
F = 128
NC, NS, L = 2, 16, 16
NW = NC * NS
NPAD = 10240
RPT = NPAD // NS
EPW = 10240
EPAD = NW * EPW
CH = 128
NCHUNK = EPW // CH
HC = NCHUNK // 2
BLK = 1024
GRID = NPAD // BLK

_MESH = plsc.VectorSubcoreMesh(
    core_axis_name="c", subcore_axis_name="s", num_cores=NC, num_subcores=NS
)


def _deg_body(src_hbm, ones_hbm, out_hbm, shared, sidx, onesb, zbuf):
    c = lax.axis_index("c")
    s = lax.axis_index("s")
    w = c * NS + s
    pltpu.sync_copy(ones_hbm.at[pl.ds(0, CH)], onesb)
    pltpu.sync_copy(ones_hbm.at[pl.ds(CH, CH)], zbuf)
    for k in range(RPT // CH):
        pltpu.sync_copy(zbuf, shared.at[pl.ds(s * RPT + k * CH, CH)])
    plsc.subcore_barrier()

    pltpu.sync_copy(src_hbm.at[w], sidx)

    def body(j, _):
        pltpu.sync_copy(onesb, shared.at[sidx.at[j]], add=True)
        return 0

    lax.fori_loop(0, NCHUNK, body, 0)
    plsc.subcore_barrier()

    for k in range(RPT // CH):
        pltpu.sync_copy(shared.at[pl.ds(s * RPT + k * CH, CH)], zbuf)
        pltpu.sync_copy(zbuf, out_hbm.at[c, pl.ds(s * RPT + k * CH, CH)])


def _scatter_body(z_hbm, src_hbm, dst_hbm, out_hbm, shared, sidx, didx,
                  gbuf0, gbuf1, sem0, sem1):
    c = lax.axis_index("c")
    s = lax.axis_index("s")
    w = c * NS + s

    pltpu.sync_copy(z_hbm.at[pl.ds(NPAD - CH, CH)], gbuf0)
    for k in range(RPT // CH):
        pltpu.sync_copy(gbuf0, shared.at[pl.ds(s * RPT + k * CH, CH)])
    plsc.subcore_barrier()

    for h in range(2):
        pltpu.sync_copy(src_hbm.at[w, pl.ds(h * HC, HC)], sidx)
        pltpu.sync_copy(dst_hbm.at[w, pl.ds(h * HC, HC)], didx)
        pltpu.async_copy(z_hbm.at[sidx.at[0]], gbuf0, sem0)

        def body(t, _):
            j0 = 2 * t
            pltpu.make_async_copy(z_hbm.at[sidx.at[j0]], gbuf0, sem0).wait()
            pltpu.async_copy(z_hbm.at[sidx.at[j0 + 1]], gbuf1, sem1)
            pltpu.sync_copy(gbuf0, shared.at[didx.at[j0]], add=True)
            pltpu.make_async_copy(z_hbm.at[sidx.at[j0 + 1]], gbuf1, sem1).wait()
            pltpu.async_copy(z_hbm.at[sidx.at[j0 + 2]], gbuf0, sem0)
            pltpu.sync_copy(gbuf1, shared.at[didx.at[j0 + 1]], add=True)
            return 0

        lax.fori_loop(0, HC // 2 - 1, body, 0)
        pltpu.make_async_copy(z_hbm.at[sidx.at[HC - 2]], gbuf0, sem0).wait()
        pltpu.async_copy(z_hbm.at[sidx.at[HC - 1]], gbuf1, sem1)
        pltpu.sync_copy(gbuf0, shared.at[didx.at[HC - 2]], add=True)
        pltpu.make_async_copy(z_hbm.at[sidx.at[HC - 1]], gbuf1, sem1).wait()
        pltpu.sync_copy(gbuf1, shared.at[didx.at[HC - 1]], add=True)
    plsc.subcore_barrier()

    for k in range(RPT // CH):
        pltpu.sync_copy(shared.at[pl.ds(s * RPT + k * CH, CH)], gbuf0)
        pltpu.sync_copy(gbuf0, out_hbm.at[c, pl.ds(s * RPT + k * CH, CH)])


_deg_kernel = functools.partial(
    pl.kernel,
    out_type=jax.ShapeDtypeStruct((NC, NPAD, F), jnp.float32),
    mesh=_MESH,
    scratch_types=[
        pltpu.VMEM_SHARED((NPAD, F), jnp.float32),
        pltpu.VMEM((NCHUNK, CH), jnp.int32),
        pltpu.VMEM((CH, F), jnp.float32),
        pltpu.VMEM((CH, F), jnp.float32),
    ],
)(_deg_body)

_scatter_kernel = functools.partial(
    pl.kernel,
    out_type=jax.ShapeDtypeStruct((NC, NPAD, F), jnp.float32),
    mesh=_MESH,
    scratch_types=[
        pltpu.VMEM_SHARED((NPAD, F), jnp.float32),
        pltpu.VMEM((HC, CH), jnp.int32),
        pltpu.VMEM((HC, CH), jnp.int32),
        pltpu.VMEM((CH, F), jnp.float32),
        pltpu.VMEM((CH, F), jnp.float32),
        pltpu.SemaphoreType.DMA,
        pltpu.SemaphoreType.DMA,
    ],
)(_scatter_body)


def _col0(m):
    e0 = (lax.broadcasted_iota(jnp.int32, (F, 1), 0) == 0).astype(jnp.float32)
    return lax.dot_general(m, e0, (((1,), (0,)), ((), ())),
                           preferred_element_type=jnp.float32)


def _pre_body(dega_ref, degb_ref, x_ref, w0_ref, w1_ref,
              xw0_ref, z1_ref, dinv_ref):
    deg = _col0(dega_ref[...] + degb_ref[...])
    rows = lax.broadcasted_iota(jnp.int32, (BLK, 1), 0) + pl.program_id(0) * BLK
    valid = (deg > 0.0) & (rows < N)
    dinv = jnp.where(valid, lax.rsqrt(jnp.maximum(deg, 1.0)), 0.0)
    x = x_ref[...]
    xw0_ref[...] = jnp.dot(x, w0_ref[...], preferred_element_type=jnp.float32)
    z1_ref[...] = jnp.dot(dinv * x, w1_ref[...], preferred_element_type=jnp.float32)
    dinv_ref[...] = dinv


def _mid_body(xw0_ref, ga_ref, gb_ref, dinv_ref, b_ref, w0_ref, w1_ref,
              h1_ref, hw0_ref, z2_ref):
    dinv = dinv_ref[...]
    h1 = jnp.maximum(
        xw0_ref[...] - dinv * (ga_ref[...] + gb_ref[...]) + b_ref[...], 0.0)
    h1_ref[...] = h1
    hw0_ref[...] = jnp.dot(h1, w0_ref[...], preferred_element_type=jnp.float32)
    z2_ref[...] = jnp.dot(dinv * h1, w1_ref[...], preferred_element_type=jnp.float32)


def _post_body(h1_ref, hw0_ref, ga_ref, gb_ref, dinv_ref, b_ref, out_ref):
    h2 = jnp.maximum(
        hw0_ref[...] - dinv_ref[...] * (ga_ref[...] + gb_ref[...]) + b_ref[...], 0.0)
    out_ref[...] = (h1_ref[...] + h2) * 0.5


_row = pl.BlockSpec((BLK, F), lambda i: (i, 0))
_rowcol = pl.BlockSpec((BLK, 1), lambda i: (i, 0))
_rowdeg = pl.BlockSpec((BLK, F), lambda i: (i, 0))
_wspec = pl.BlockSpec((F, F), lambda i: (0, 0))
_bspec = pl.BlockSpec((1, F), lambda i: (0, 0))

_mat = jax.ShapeDtypeStruct((NPAD, F), jnp.float32)
_colv = jax.ShapeDtypeStruct((NPAD, 1), jnp.float32)

_pre_call = pl.pallas_call(
    _pre_body,
    grid=(GRID,),
    in_specs=[_rowdeg, _rowdeg, _row, _wspec, _wspec],
    out_specs=[_row, _row, _rowcol],
    out_shape=[_mat, _mat, _colv],
)

_mid_call = pl.pallas_call(
    _mid_body,
    grid=(GRID,),
    in_specs=[_row, _row, _row, _rowcol, _bspec, _wspec, _wspec],
    out_specs=[_row, _row, _row],
    out_shape=[_mat, _mat, _mat],
)

_post_call = pl.pallas_call(
    _post_body,
    grid=(GRID,),
    in_specs=[_row, _row, _row, _row, _rowcol, _bspec],
    out_specs=_row,
    out_shape=_mat,
)


def kernel(x, edge_index, W0_1, W1_1, b1, W0_2, W1_2, b2):
    src = edge_index[0]
    dst = edge_index[1]
    e = src.shape[0]
    pad_e = EPAD - e
    srcp = jnp.concatenate([src, jnp.full((pad_e,), N, jnp.int32)])
    dstp = jnp.concatenate([dst, jnp.full((pad_e,), NPAD - 1, jnp.int32)])
    src_b = srcp.reshape(NW, NCHUNK, CH)
    dst_b = dstp.reshape(NW, NCHUNK, CH)
    xp = jnp.concatenate([x, jnp.zeros((NPAD - N, F), jnp.float32)], axis=0)

    ones_c = jnp.zeros((2 * CH, F), jnp.float32).at[:CH, 0].set(1.0)
    degp = _deg_kernel(src_b, ones_c)
    xw0, z1, dinv = _pre_call(degp[0], degp[1], xp, W0_1, W1_1)
    g1 = _scatter_kernel(z1, src_b, dst_b)
    h1, hw0, z2 = _mid_call(xw0, g1[0], g1[1], dinv, b1.reshape(1, F), W0_2, W1_2)
    g2 = _scatter_kernel(z2, src_b, dst_b)
    out = _post_call(h1, hw0, g2[0], g2[1], dinv, b2.reshape(1, F))
    return out[:N]
